# Initial kernel scaffold; baseline (speedup 1.0000x reference)
import jax
import jax.numpy as jnp
from jax.experimental import pallas as pl


def kernel(x, W_fc, b_fc, W1a, b1a, W1b, b1b, W2a, b2a, W2b, b2b, Wa, ba, Wb, bb, Wc, bc, Wr, br, Wk, bk, edge_index, batch):
    raise NotImplementedError("write your pallas kernel here")


# speedup vs baseline: 4.3302x; 4.3302x over previous
import functools

import jax
import jax.numpy as jnp
from jax import lax
from jax.experimental import pallas as pl
from jax.experimental.pallas import tpu as pltpu
from jax.experimental.pallas import tpu_sc as plsc

_N = 10000
_E = 320000
_B = 8
_DIN = 1792
_H = 128
_H3 = 384

_NACC = 10240
_NSUB = 16
_NCORE = 2
_EPW = 10240
_EPAD = _EPW * _NSUB * _NCORE
_K = 128
_NCH = _EPW // _K
_RPS = _NACC // _NSUB
_RCH = 128
_NRC = _RPS // _RCH

_BN = 2048
_G = 5


def _build_agg():
    mesh = plsc.VectorSubcoreMesh(core_axis_name="c", subcore_axis_name="s")

    @functools.partial(
        pl.kernel,
        mesh=mesh,
        out_type=[
            jax.ShapeDtypeStruct((_NACC, _H), jnp.float32),
            jax.ShapeDtypeStruct((_NACC, _H), jnp.float32),
        ],
        scratch_types=[
            pltpu.VMEM((_K,), jnp.int32),
            pltpu.VMEM((_K,), jnp.int32),
            pltpu.VMEM((_K, _H), jnp.float32),
            pltpu.VMEM((_RCH, _H), jnp.float32),
            pltpu.VMEM_SHARED((_NACC, _H), jnp.float32),
            pltpu.SemaphoreType.DMA,
        ],
    )
    def agg(h_hbm, src_hbm, dst_hbm, out0_hbm, out1_hbm,
            src_v, dst_v, rows_v, tmp_v, acc_sh, sem):
        cid = lax.axis_index("c")
        sid = lax.axis_index("s")
        wid = cid * _NSUB + sid

        def _zrow(r, carry):
            for cc in range(_H // 16):
                tmp_v[r, pl.ds(cc * 16, 16)] = jnp.zeros((16,), jnp.float32)
            return carry

        lax.fori_loop(0, _RCH, _zrow, 0)
        for j in range(_NRC):
            pltpu.sync_copy(tmp_v, acc_sh.at[pl.ds(sid * _RPS + j * _RCH, _RCH)])
        plsc.subcore_barrier()

        base = wid * _EPW

        def _chunk(c, carry):
            off = base + c * _K
            pltpu.sync_copy(src_hbm.at[pl.ds(off, _K)], src_v)
            pltpu.sync_copy(dst_hbm.at[pl.ds(off, _K)], dst_v)
            pltpu.async_copy(h_hbm.at[src_v], rows_v, sem).wait()
            pltpu.sync_copy(rows_v, acc_sh.at[dst_v], add=True)
            return carry

        lax.fori_loop(0, _NCH, _chunk, 0)
        plsc.subcore_barrier()

        for j in range(_NRC):
            r0 = sid * _RPS + j * _RCH
            pltpu.sync_copy(acc_sh.at[pl.ds(r0, _RCH)], tmp_v)

            @pl.when(cid == 0)
            def _():
                pltpu.sync_copy(tmp_v, out0_hbm.at[pl.ds(r0, _RCH)])

            @pl.when(cid == 1)
            def _():
                pltpu.sync_copy(tmp_v, out1_hbm.at[pl.ds(r0, _RCH)])

    return agg


_agg_cache = []


def _get_agg():
    if not _agg_cache:
        _agg_cache.append(_build_agg())
    return _agg_cache[0]


def _fc_body(x_r, w_r, b_r, o_r):
    o_r[...] = jnp.maximum(
        jnp.dot(x_r[...], w_r[...], preferred_element_type=jnp.float32, precision=lax.Precision.HIGHEST) + b_r[...],
        0.0)


def _fc_call(x, W, b):
    return pl.pallas_call(
        _fc_body,
        grid=(_G,),
        in_specs=[
            pl.BlockSpec((_BN, _DIN), lambda i: (i, 0)),
            pl.BlockSpec((_DIN, _H), lambda i: (0, 0)),
            pl.BlockSpec((1, _H), lambda i: (0, 0)),
        ],
        out_specs=pl.BlockSpec((_BN, _H), lambda i: (i, 0)),
        out_shape=jax.ShapeDtypeStruct((_N, _H), jnp.float32),
    )(x, W, b)


def _gin_body(h_r, a0_r, a1_r, w1_r, b1_r, w2_r, b2_r, o_r):
    z = h_r[...] + a0_r[...] + a1_r[...]
    t = jnp.maximum(
        jnp.dot(z, w1_r[...], preferred_element_type=jnp.float32, precision=lax.Precision.HIGHEST) + b1_r[...], 0.0)
    o_r[...] = jnp.maximum(
        jnp.dot(t, w2_r[...], preferred_element_type=jnp.float32, precision=lax.Precision.HIGHEST) + b2_r[...], 0.0)


def _gin_call(h, a0, a1, W1, b1, W2, b2):
    return pl.pallas_call(
        _gin_body,
        grid=(_G,),
        in_specs=[
            pl.BlockSpec((_BN, _H), lambda i: (i, 0)),
            pl.BlockSpec((_BN, _H), lambda i: (i, 0)),
            pl.BlockSpec((_BN, _H), lambda i: (i, 0)),
            pl.BlockSpec((_H, _H), lambda i: (0, 0)),
            pl.BlockSpec((1, _H), lambda i: (0, 0)),
            pl.BlockSpec((_H, _H), lambda i: (0, 0)),
            pl.BlockSpec((1, _H), lambda i: (0, 0)),
        ],
        out_specs=pl.BlockSpec((_BN, _H), lambda i: (i, 0)),
        out_shape=jax.ShapeDtypeStruct((_N, _H), jnp.float32),
    )(h, a0, a1, W1, b1, W2, b2)


def _att_body(h0_r, h1_r, h2_r, bt_r, wa_r, ba_r, wb_r, bb_r, wc_r, bc_r,
              a_out_r, m_out_r):
    i = pl.program_id(0)
    hp = jnp.concatenate([h0_r[...], h1_r[...], h2_r[...]], axis=1)
    a = jnp.tanh(jnp.dot(hp, wa_r[...], preferred_element_type=jnp.float32, precision=lax.Precision.HIGHEST)
                 + ba_r[...])
    g = jax.nn.sigmoid(jnp.dot(hp, wb_r[...], preferred_element_type=jnp.float32, precision=lax.Precision.HIGHEST)
                       + bb_r[...])
    A = jnp.dot(a * g, wc_r[...], preferred_element_type=jnp.float32, precision=lax.Precision.HIGHEST) + bc_r[...]
    a_out_r[...] = A
    rows = lax.broadcasted_iota(jnp.int32, (_BN, 1), 0) + i * _BN
    valid = rows < _N
    seg = lax.broadcasted_iota(jnp.int32, (_BN, _B), 1)
    mask = (bt_r[...] == seg) & valid
    vals = jnp.where(mask, A, -jnp.inf)
    blkmax = jnp.max(vals, axis=0, keepdims=True)

    @pl.when(i == 0)
    def _():
        m_out_r[...] = jnp.full((1, _B), -jnp.inf, jnp.float32)

    m_out_r[...] = jnp.maximum(m_out_r[...], blkmax)


def _att_call(h0, h1, h2, batch2, Wa, ba, Wb, bb, Wc, bc):
    return pl.pallas_call(
        _att_body,
        grid=(_G,),
        in_specs=[
            pl.BlockSpec((_BN, _H), lambda i: (i, 0)),
            pl.BlockSpec((_BN, _H), lambda i: (i, 0)),
            pl.BlockSpec((_BN, _H), lambda i: (i, 0)),
            pl.BlockSpec((_BN, 1), lambda i: (i, 0)),
            pl.BlockSpec((_H3, _H3), lambda i: (0, 0)),
            pl.BlockSpec((1, _H3), lambda i: (0, 0)),
            pl.BlockSpec((_H3, _H3), lambda i: (0, 0)),
            pl.BlockSpec((1, _H3), lambda i: (0, 0)),
            pl.BlockSpec((_H3, 1), lambda i: (0, 0)),
            pl.BlockSpec((1, 1), lambda i: (0, 0)),
        ],
        out_specs=[
            pl.BlockSpec((_BN, 1), lambda i: (i, 0)),
            pl.BlockSpec((1, _B), lambda i: (0, 0)),
        ],
        out_shape=[
            jax.ShapeDtypeStruct((_N, 1), jnp.float32),
            jax.ShapeDtypeStruct((1, _B), jnp.float32),
        ],
    )(h0, h1, h2, batch2, Wa, ba, Wb, bb, Wc, bc)


def _pool_body(h0_r, h1_r, h2_r, bt_r, a_r, m_r, wr_r, br_r, wk_r, bk_r,
               o_r, u_acc, s_acc):
    i = pl.program_id(0)

    @pl.when(i == 0)
    def _():
        u_acc[...] = jnp.zeros((_B, _H3), jnp.float32)
        s_acc[...] = jnp.zeros((_B, 1), jnp.float32)

    rows = lax.broadcasted_iota(jnp.int32, (_BN, 1), 0) + i * _BN
    valid = rows < _N
    hp = jnp.concatenate([h0_r[...], h1_r[...], h2_r[...]], axis=1)
    hp = jnp.where(valid, hp, 0.0)
    seg = lax.broadcasted_iota(jnp.int32, (_BN, _B), 1)
    mask = (bt_r[...] == seg) & valid
    e = jnp.where(mask, jnp.exp(a_r[...] - m_r[...]), 0.0)
    u_acc[...] += lax.dot_general(
        e, hp, (((0,), (0,)), ((), ())), preferred_element_type=jnp.float32, precision=lax.Precision.HIGHEST)
    s_acc[...] += lax.dot_general(
        e, jnp.ones((_BN, 1), jnp.float32), (((0,), (0,)), ((), ())),
        preferred_element_type=jnp.float32, precision=lax.Precision.HIGHEST)

    @pl.when(i == _G - 1)
    def _():
        s = s_acc[...]
        s_safe = jnp.where(s > 0.0, s, 1.0)
        pooled = u_acc[...] / s_safe
        hr = jnp.maximum(
            jnp.dot(pooled, wr_r[...], preferred_element_type=jnp.float32, precision=lax.Precision.HIGHEST)
            + br_r[...], 0.0)
        o_r[...] = (jnp.dot(hr, wk_r[...], preferred_element_type=jnp.float32, precision=lax.Precision.HIGHEST)
                    + bk_r[...])


def _pool_call(h0, h1, h2, batch2, A, m, Wr, br, Wk, bk):
    return pl.pallas_call(
        _pool_body,
        grid=(_G,),
        in_specs=[
            pl.BlockSpec((_BN, _H), lambda i: (i, 0)),
            pl.BlockSpec((_BN, _H), lambda i: (i, 0)),
            pl.BlockSpec((_BN, _H), lambda i: (i, 0)),
            pl.BlockSpec((_BN, 1), lambda i: (i, 0)),
            pl.BlockSpec((_BN, 1), lambda i: (i, 0)),
            pl.BlockSpec((1, _B), lambda i: (0, 0)),
            pl.BlockSpec((_H3, _H3), lambda i: (0, 0)),
            pl.BlockSpec((1, _H3), lambda i: (0, 0)),
            pl.BlockSpec((_H3, 1), lambda i: (0, 0)),
            pl.BlockSpec((1, 1), lambda i: (0, 0)),
        ],
        out_specs=pl.BlockSpec((_B, 1), lambda i: (0, 0)),
        out_shape=jax.ShapeDtypeStruct((_B, 1), jnp.float32),
        scratch_shapes=[
            pltpu.VMEM((_B, _H3), jnp.float32),
            pltpu.VMEM((_B, 1), jnp.float32),
        ],
    )(h0, h1, h2, batch2, A, m, Wr, br, Wk, bk)


def kernel(x, W_fc, b_fc, W1a, b1a, W1b, b1b, W2a, b2a, W2b, b2b,
           Wa, ba, Wb, bb, Wc, bc, Wr, br, Wk, bk, edge_index, batch):
    src = edge_index[0]
    dst = edge_index[1]
    pad = _EPAD - _E
    pi = jnp.arange(pad, dtype=jnp.int32)
    src_p = jnp.concatenate([src, pi % _N])
    dst_p = jnp.concatenate([dst, _N + pi % (_NACC - _N)])
    batch2 = batch.reshape(_N, 1)

    agg = _get_agg()
    h0 = _fc_call(x, W_fc, b_fc.reshape(1, _H))
    a00, a01 = agg(h0, src_p, dst_p)
    h1 = _gin_call(h0, a00, a01, W1a, b1a.reshape(1, _H), W1b, b1b.reshape(1, _H))
    a10, a11 = agg(h1, src_p, dst_p)
    h2 = _gin_call(h1, a10, a11, W2a, b2a.reshape(1, _H), W2b, b2b.reshape(1, _H))

    A, m = _att_call(h0, h1, h2, batch2,
                     Wa, ba.reshape(1, _H3), Wb, bb.reshape(1, _H3),
                     Wc, bc.reshape(1, 1))
    out = _pool_call(h0, h1, h2, batch2, A, m,
                     Wr, br.reshape(1, _H3), Wk, bk.reshape(1, 1))
    return out.reshape(-1)


# speedup vs baseline: 6.5804x; 1.5197x over previous
import functools

import jax
import jax.numpy as jnp
from jax import lax
from jax.experimental import pallas as pl
from jax.experimental.pallas import tpu as pltpu
from jax.experimental.pallas import tpu_sc as plsc

_N = 10000
_E = 320000
_B = 8
_DIN = 1792
_H = 128
_H3 = 384

_NACC = 10240
_NSUB = 16
_NCORE = 2
_EPW = 10240
_EPAD = _EPW * _NSUB * _NCORE
_K = 128
_NCH = _EPW // _K
_NPAIR = _NCH // 2
_RPS = _NACC // _NSUB
_RCH = 128
_NRC = _RPS // _RCH

_BN = 2048
_G = 5


def _build_agg():
    mesh = plsc.VectorSubcoreMesh(core_axis_name="c", subcore_axis_name="s")

    @functools.partial(
        pl.kernel,
        mesh=mesh,
        out_type=[
            jax.ShapeDtypeStruct((_NACC, _H), jnp.float32),
            jax.ShapeDtypeStruct((_NACC, _H), jnp.float32),
        ],
        scratch_types=[
            pltpu.VMEM((1, _K), jnp.int32),
            pltpu.VMEM((1, _K), jnp.int32),
            pltpu.VMEM((1, _K), jnp.int32),
            pltpu.VMEM((1, _K), jnp.int32),
            pltpu.VMEM((_K, _H), jnp.float32),
            pltpu.VMEM((_K, _H), jnp.float32),
            pltpu.VMEM_SHARED((_NACC, _H), jnp.float32),
            pltpu.SemaphoreType.DMA,
            pltpu.SemaphoreType.DMA,
            pltpu.SemaphoreType.DMA,
            pltpu.SemaphoreType.DMA,
        ],
    )
    def agg(h_hbm, src_hbm, dst_hbm, out0_hbm, out1_hbm,
            si0, si1, di0, di1, rows0, rows1, acc_sh,
            semi0, semi1, semg0, semg1):
        cid = lax.axis_index("c")
        sid = lax.axis_index("s")
        wid = cid * _NSUB + sid
        base = wid * _NCH

        pltpu.async_copy(src_hbm.at[pl.ds(base, 1)], si0, semi0)
        pltpu.async_copy(dst_hbm.at[pl.ds(base, 1)], di0, semi0)

        def _zrow(r, carry):
            for cc in range(_H // 16):
                rows0[r, pl.ds(cc * 16, 16)] = jnp.zeros((16,), jnp.float32)
            return carry

        lax.fori_loop(0, _RCH, _zrow, 0)
        for j in range(_NRC):
            pltpu.sync_copy(rows0, acc_sh.at[pl.ds(sid * _RPS + j * _RCH, _RCH)])
        plsc.subcore_barrier()

        def _ichunk(c, sv, dv, sem):
            pltpu.async_copy(src_hbm.at[pl.ds(base + c, 1)], sv, sem)
            pltpu.async_copy(dst_hbm.at[pl.ds(base + c, 1)], dv, sem)

        def _iwait(c, sv, dv, sem):
            pltpu.make_async_copy(src_hbm.at[pl.ds(base + c, 1)], sv, sem).wait()
            pltpu.make_async_copy(dst_hbm.at[pl.ds(base + c, 1)], dv, sem).wait()

        def _pair(p, carry):
            c0 = 2 * p
            c1 = c0 + 1
            _iwait(c0, si0, di0, semi0)
            pltpu.async_copy(h_hbm.at[si0.at[0]], rows0, semg0)

            @pl.when(p > 0)
            def _():
                pltpu.make_async_copy(h_hbm.at[si1.at[0]], rows1, semg1).wait()
                pltpu.sync_copy(rows1, acc_sh.at[di1.at[0]], add=True)

            _ichunk(c1, si1, di1, semi1)
            _iwait(c1, si1, di1, semi1)
            pltpu.async_copy(h_hbm.at[si1.at[0]], rows1, semg1)
            pltpu.make_async_copy(h_hbm.at[si0.at[0]], rows0, semg0).wait()
            pltpu.sync_copy(rows0, acc_sh.at[di0.at[0]], add=True)

            @pl.when(p + 1 < _NPAIR)
            def _():
                _ichunk(c0 + 2, si0, di0, semi0)

            return carry

        lax.fori_loop(0, _NPAIR, _pair, 0)
        pltpu.make_async_copy(h_hbm.at[si1.at[0]], rows1, semg1).wait()
        pltpu.sync_copy(rows1, acc_sh.at[di1.at[0]], add=True)
        plsc.subcore_barrier()

        for j in range(_NRC):
            r0 = sid * _RPS + j * _RCH
            pltpu.sync_copy(acc_sh.at[pl.ds(r0, _RCH)], rows0)

            @pl.when(cid == 0)
            def _():
                pltpu.sync_copy(rows0, out0_hbm.at[pl.ds(r0, _RCH)])

            @pl.when(cid == 1)
            def _():
                pltpu.sync_copy(rows0, out1_hbm.at[pl.ds(r0, _RCH)])

    return agg


_agg_cache = []


def _get_agg():
    if not _agg_cache:
        _agg_cache.append(_build_agg())
    return _agg_cache[0]


def _fc_body(x_r, w_r, b_r, o_r):
    o_r[...] = jnp.maximum(
        jnp.dot(x_r[...], w_r[...], preferred_element_type=jnp.float32, precision=lax.Precision.HIGHEST) + b_r[...],
        0.0)


def _fc_call(x, W, b):
    return pl.pallas_call(
        _fc_body,
        grid=(_G,),
        in_specs=[
            pl.BlockSpec((_BN, _DIN), lambda i: (i, 0)),
            pl.BlockSpec((_DIN, _H), lambda i: (0, 0)),
            pl.BlockSpec((1, _H), lambda i: (0, 0)),
        ],
        out_specs=pl.BlockSpec((_BN, _H), lambda i: (i, 0)),
        out_shape=jax.ShapeDtypeStruct((_N, _H), jnp.float32),
    )(x, W, b)


def _gin_body(h_r, a0_r, a1_r, w1_r, b1_r, w2_r, b2_r, o_r):
    z = h_r[...] + a0_r[...] + a1_r[...]
    t = jnp.maximum(
        jnp.dot(z, w1_r[...], preferred_element_type=jnp.float32, precision=lax.Precision.HIGHEST) + b1_r[...], 0.0)
    o_r[...] = jnp.maximum(
        jnp.dot(t, w2_r[...], preferred_element_type=jnp.float32, precision=lax.Precision.HIGHEST) + b2_r[...], 0.0)


def _gin_call(h, a0, a1, W1, b1, W2, b2):
    return pl.pallas_call(
        _gin_body,
        grid=(_G,),
        in_specs=[
            pl.BlockSpec((_BN, _H), lambda i: (i, 0)),
            pl.BlockSpec((_BN, _H), lambda i: (i, 0)),
            pl.BlockSpec((_BN, _H), lambda i: (i, 0)),
            pl.BlockSpec((_H, _H), lambda i: (0, 0)),
            pl.BlockSpec((1, _H), lambda i: (0, 0)),
            pl.BlockSpec((_H, _H), lambda i: (0, 0)),
            pl.BlockSpec((1, _H), lambda i: (0, 0)),
        ],
        out_specs=pl.BlockSpec((_BN, _H), lambda i: (i, 0)),
        out_shape=jax.ShapeDtypeStruct((_N, _H), jnp.float32),
    )(h, a0, a1, W1, b1, W2, b2)


def _att_body(h0_r, h1_r, h2_r, bt_r, wa_r, ba_r, wb_r, bb_r, wc_r, bc_r,
              a_out_r, m_out_r):
    i = pl.program_id(0)
    hp = jnp.concatenate([h0_r[...], h1_r[...], h2_r[...]], axis=1)
    a = jnp.tanh(jnp.dot(hp, wa_r[...], preferred_element_type=jnp.float32, precision=lax.Precision.HIGHEST)
                 + ba_r[...])
    g = jax.nn.sigmoid(jnp.dot(hp, wb_r[...], preferred_element_type=jnp.float32, precision=lax.Precision.HIGHEST)
                       + bb_r[...])
    A = jnp.dot(a * g, wc_r[...], preferred_element_type=jnp.float32, precision=lax.Precision.HIGHEST) + bc_r[...]
    a_out_r[...] = A
    rows = lax.broadcasted_iota(jnp.int32, (_BN, 1), 0) + i * _BN
    valid = rows < _N
    seg = lax.broadcasted_iota(jnp.int32, (_BN, _B), 1)
    mask = (bt_r[...] == seg) & valid
    vals = jnp.where(mask, A, -jnp.inf)
    blkmax = jnp.max(vals, axis=0, keepdims=True)

    @pl.when(i == 0)
    def _():
        m_out_r[...] = jnp.full((1, _B), -jnp.inf, jnp.float32)

    m_out_r[...] = jnp.maximum(m_out_r[...], blkmax)


def _att_call(h0, h1, h2, batch2, Wa, ba, Wb, bb, Wc, bc):
    return pl.pallas_call(
        _att_body,
        grid=(_G,),
        in_specs=[
            pl.BlockSpec((_BN, _H), lambda i: (i, 0)),
            pl.BlockSpec((_BN, _H), lambda i: (i, 0)),
            pl.BlockSpec((_BN, _H), lambda i: (i, 0)),
            pl.BlockSpec((_BN, 1), lambda i: (i, 0)),
            pl.BlockSpec((_H3, _H3), lambda i: (0, 0)),
            pl.BlockSpec((1, _H3), lambda i: (0, 0)),
            pl.BlockSpec((_H3, _H3), lambda i: (0, 0)),
            pl.BlockSpec((1, _H3), lambda i: (0, 0)),
            pl.BlockSpec((_H3, 1), lambda i: (0, 0)),
            pl.BlockSpec((1, 1), lambda i: (0, 0)),
        ],
        out_specs=[
            pl.BlockSpec((_BN, 1), lambda i: (i, 0)),
            pl.BlockSpec((1, _B), lambda i: (0, 0)),
        ],
        out_shape=[
            jax.ShapeDtypeStruct((_N, 1), jnp.float32),
            jax.ShapeDtypeStruct((1, _B), jnp.float32),
        ],
    )(h0, h1, h2, batch2, Wa, ba, Wb, bb, Wc, bc)


def _pool_body(h0_r, h1_r, h2_r, bt_r, a_r, m_r, wr_r, br_r, wk_r, bk_r,
               o_r, u_acc, s_acc):
    i = pl.program_id(0)

    @pl.when(i == 0)
    def _():
        u_acc[...] = jnp.zeros((_B, _H3), jnp.float32)
        s_acc[...] = jnp.zeros((_B, 1), jnp.float32)

    rows = lax.broadcasted_iota(jnp.int32, (_BN, 1), 0) + i * _BN
    valid = rows < _N
    hp = jnp.concatenate([h0_r[...], h1_r[...], h2_r[...]], axis=1)
    hp = jnp.where(valid, hp, 0.0)
    seg = lax.broadcasted_iota(jnp.int32, (_BN, _B), 1)
    mask = (bt_r[...] == seg) & valid
    e = jnp.where(mask, jnp.exp(a_r[...] - m_r[...]), 0.0)
    u_acc[...] += lax.dot_general(
        e, hp, (((0,), (0,)), ((), ())), preferred_element_type=jnp.float32, precision=lax.Precision.HIGHEST)
    s_acc[...] += lax.dot_general(
        e, jnp.ones((_BN, 1), jnp.float32), (((0,), (0,)), ((), ())),
        preferred_element_type=jnp.float32, precision=lax.Precision.HIGHEST)

    @pl.when(i == _G - 1)
    def _():
        s = s_acc[...]
        s_safe = jnp.where(s > 0.0, s, 1.0)
        pooled = u_acc[...] / s_safe
        hr = jnp.maximum(
            jnp.dot(pooled, wr_r[...], preferred_element_type=jnp.float32, precision=lax.Precision.HIGHEST)
            + br_r[...], 0.0)
        o_r[...] = (jnp.dot(hr, wk_r[...], preferred_element_type=jnp.float32, precision=lax.Precision.HIGHEST)
                    + bk_r[...])


def _pool_call(h0, h1, h2, batch2, A, m, Wr, br, Wk, bk):
    return pl.pallas_call(
        _pool_body,
        grid=(_G,),
        in_specs=[
            pl.BlockSpec((_BN, _H), lambda i: (i, 0)),
            pl.BlockSpec((_BN, _H), lambda i: (i, 0)),
            pl.BlockSpec((_BN, _H), lambda i: (i, 0)),
            pl.BlockSpec((_BN, 1), lambda i: (i, 0)),
            pl.BlockSpec((_BN, 1), lambda i: (i, 0)),
            pl.BlockSpec((1, _B), lambda i: (0, 0)),
            pl.BlockSpec((_H3, _H3), lambda i: (0, 0)),
            pl.BlockSpec((1, _H3), lambda i: (0, 0)),
            pl.BlockSpec((_H3, 1), lambda i: (0, 0)),
            pl.BlockSpec((1, 1), lambda i: (0, 0)),
        ],
        out_specs=pl.BlockSpec((_B, 1), lambda i: (0, 0)),
        out_shape=jax.ShapeDtypeStruct((_B, 1), jnp.float32),
        scratch_shapes=[
            pltpu.VMEM((_B, _H3), jnp.float32),
            pltpu.VMEM((_B, 1), jnp.float32),
        ],
    )(h0, h1, h2, batch2, A, m, Wr, br, Wk, bk)


def kernel(x, W_fc, b_fc, W1a, b1a, W1b, b1b, W2a, b2a, W2b, b2b,
           Wa, ba, Wb, bb, Wc, bc, Wr, br, Wk, bk, edge_index, batch):
    src = edge_index[0]
    dst = edge_index[1]
    pad = _EPAD - _E
    pi = jnp.arange(pad, dtype=jnp.int32)
    src_p = jnp.concatenate([src, pi % _N]).reshape(_EPAD // _K, _K)
    dst_p = jnp.concatenate([dst, _N + pi % (_NACC - _N)]).reshape(_EPAD // _K, _K)
    batch2 = batch.reshape(_N, 1)

    agg = _get_agg()
    h0 = _fc_call(x, W_fc, b_fc.reshape(1, _H))
    a00, a01 = agg(h0, src_p, dst_p)
    h1 = _gin_call(h0, a00, a01, W1a, b1a.reshape(1, _H), W1b, b1b.reshape(1, _H))
    a10, a11 = agg(h1, src_p, dst_p)
    h2 = _gin_call(h1, a10, a11, W2a, b2a.reshape(1, _H), W2b, b2b.reshape(1, _H))

    A, m = _att_call(h0, h1, h2, batch2,
                     Wa, ba.reshape(1, _H3), Wb, bb.reshape(1, _H3),
                     Wc, bc.reshape(1, 1))
    out = _pool_call(h0, h1, h2, batch2, A, m,
                     Wr, br.reshape(1, _H3), Wk, bk.reshape(1, 1))
    return out.reshape(-1)


# speedup vs baseline: 7.3150x; 1.1116x over previous
import functools

import jax
import jax.numpy as jnp
from jax import lax
from jax.experimental import pallas as pl
from jax.experimental.pallas import tpu as pltpu
from jax.experimental.pallas import tpu_sc as plsc

_N = 10000
_E = 320000
_B = 8
_DIN = 1792
_H = 128
_H3 = 384

_NACC = 10240
_NSUB = 16
_NCORE = 2
_EPW = 10240
_EPAD = _EPW * _NSUB * _NCORE
_K = 128
_NCH = _EPW // _K
_NPAIR = _NCH // 2
_RPS = _NACC // _NSUB
_RCH = 128
_NRC = _RPS // _RCH

_BN = 2048
_G = 5


def _build_agg():
    mesh = plsc.VectorSubcoreMesh(core_axis_name="c", subcore_axis_name="s")

    @functools.partial(
        pl.kernel,
        mesh=mesh,
        out_type=[
            jax.ShapeDtypeStruct((_NACC, _H), jnp.float32),
            jax.ShapeDtypeStruct((_NACC, _H), jnp.float32),
        ],
        scratch_types=[
            pltpu.VMEM((1, _K), jnp.int32),
            pltpu.VMEM((1, _K), jnp.int32),
            pltpu.VMEM((1, _K), jnp.int32),
            pltpu.VMEM((1, _K), jnp.int32),
            pltpu.VMEM((_K, _H), jnp.float32),
            pltpu.VMEM((_K, _H), jnp.float32),
            pltpu.VMEM_SHARED((_NACC, _H), jnp.float32),
            pltpu.SemaphoreType.DMA,
            pltpu.SemaphoreType.DMA,
            pltpu.SemaphoreType.DMA,
            pltpu.SemaphoreType.DMA,
        ],
    )
    def agg(h_hbm, src_hbm, dst_hbm, out0_hbm, out1_hbm,
            si0, si1, di0, di1, rows0, rows1, acc_sh,
            semi0, semi1, semg0, semg1):
        cid = lax.axis_index("c")
        sid = lax.axis_index("s")
        wid = cid * _NSUB + sid
        base = wid * _NCH

        pltpu.async_copy(src_hbm.at[pl.ds(base, 1)], si0, semi0)
        pltpu.async_copy(dst_hbm.at[pl.ds(base, 1)], di0, semi0)

        def _zrow(r, carry):
            for cc in range(_H // 16):
                rows0[r, pl.ds(cc * 16, 16)] = jnp.zeros((16,), jnp.float32)
            return carry

        lax.fori_loop(0, _RCH, _zrow, 0)
        for j in range(_NRC):
            pltpu.sync_copy(rows0, acc_sh.at[pl.ds(sid * _RPS + j * _RCH, _RCH)])
        plsc.subcore_barrier()

        def _ichunk(c, sv, dv, sem):
            pltpu.async_copy(src_hbm.at[pl.ds(base + c, 1)], sv, sem)
            pltpu.async_copy(dst_hbm.at[pl.ds(base + c, 1)], dv, sem)

        def _iwait(c, sv, dv, sem):
            pltpu.make_async_copy(src_hbm.at[pl.ds(base + c, 1)], sv, sem).wait()
            pltpu.make_async_copy(dst_hbm.at[pl.ds(base + c, 1)], dv, sem).wait()

        def _pair(p, carry):
            c0 = 2 * p
            c1 = c0 + 1
            _iwait(c0, si0, di0, semi0)
            pltpu.async_copy(h_hbm.at[si0.at[0]], rows0, semg0)

            @pl.when(p > 0)
            def _():
                pltpu.make_async_copy(h_hbm.at[si1.at[0]], rows1, semg1).wait()
                pltpu.sync_copy(rows1, acc_sh.at[di1.at[0]], add=True)

            _ichunk(c1, si1, di1, semi1)
            _iwait(c1, si1, di1, semi1)
            pltpu.async_copy(h_hbm.at[si1.at[0]], rows1, semg1)
            pltpu.make_async_copy(h_hbm.at[si0.at[0]], rows0, semg0).wait()
            pltpu.sync_copy(rows0, acc_sh.at[di0.at[0]], add=True)

            @pl.when(p + 1 < _NPAIR)
            def _():
                _ichunk(c0 + 2, si0, di0, semi0)

            return carry

        lax.fori_loop(0, _NPAIR, _pair, 0)
        pltpu.make_async_copy(h_hbm.at[si1.at[0]], rows1, semg1).wait()
        pltpu.sync_copy(rows1, acc_sh.at[di1.at[0]], add=True)
        plsc.subcore_barrier()

        for j in range(_NRC):
            r0 = sid * _RPS + j * _RCH
            pltpu.sync_copy(acc_sh.at[pl.ds(r0, _RCH)], rows0)

            @pl.when(cid == 0)
            def _():
                pltpu.sync_copy(rows0, out0_hbm.at[pl.ds(r0, _RCH)])

            @pl.when(cid == 1)
            def _():
                pltpu.sync_copy(rows0, out1_hbm.at[pl.ds(r0, _RCH)])

    return agg


_agg_cache = []


def _get_agg():
    if not _agg_cache:
        _agg_cache.append(_build_agg())
    return _agg_cache[0]


def _fc_body(x_r, w_r, b_r, o_r):
    o_r[...] = jnp.maximum(
        jnp.dot(x_r[...], w_r[...], preferred_element_type=jnp.float32) + b_r[...],
        0.0)


def _fc_call(x, W, b):
    return pl.pallas_call(
        _fc_body,
        grid=(_G,),
        in_specs=[
            pl.BlockSpec((_BN, _DIN), lambda i: (i, 0)),
            pl.BlockSpec((_DIN, _H), lambda i: (0, 0)),
            pl.BlockSpec((1, _H), lambda i: (0, 0)),
        ],
        out_specs=pl.BlockSpec((_BN, _H), lambda i: (i, 0)),
        out_shape=jax.ShapeDtypeStruct((_N, _H), jnp.float32),
    )(x, W, b)


def _gin_body(h_r, a0_r, a1_r, w1_r, b1_r, w2_r, b2_r, o_r):
    z = h_r[...] + a0_r[...] + a1_r[...]
    t = jnp.maximum(
        jnp.dot(z, w1_r[...], preferred_element_type=jnp.float32, precision=lax.Precision.HIGHEST) + b1_r[...], 0.0)
    o_r[...] = jnp.maximum(
        jnp.dot(t, w2_r[...], preferred_element_type=jnp.float32, precision=lax.Precision.HIGHEST) + b2_r[...], 0.0)


def _gin_call(h, a0, a1, W1, b1, W2, b2):
    return pl.pallas_call(
        _gin_body,
        grid=(_G,),
        in_specs=[
            pl.BlockSpec((_BN, _H), lambda i: (i, 0)),
            pl.BlockSpec((_BN, _H), lambda i: (i, 0)),
            pl.BlockSpec((_BN, _H), lambda i: (i, 0)),
            pl.BlockSpec((_H, _H), lambda i: (0, 0)),
            pl.BlockSpec((1, _H), lambda i: (0, 0)),
            pl.BlockSpec((_H, _H), lambda i: (0, 0)),
            pl.BlockSpec((1, _H), lambda i: (0, 0)),
        ],
        out_specs=pl.BlockSpec((_BN, _H), lambda i: (i, 0)),
        out_shape=jax.ShapeDtypeStruct((_N, _H), jnp.float32),
    )(h, a0, a1, W1, b1, W2, b2)


def _att_body(h0_r, h1_r, h2_r, bt_r, wa_r, ba_r, wb_r, bb_r, wc_r, bc_r,
              a_out_r, m_out_r):
    i = pl.program_id(0)
    hp = jnp.concatenate([h0_r[...], h1_r[...], h2_r[...]], axis=1)
    a = jnp.tanh(jnp.dot(hp, wa_r[...], preferred_element_type=jnp.float32,
                         precision=lax.Precision.HIGHEST) + ba_r[...])
    g = jax.nn.sigmoid(jnp.dot(hp, wb_r[...], preferred_element_type=jnp.float32,
                               precision=lax.Precision.HIGHEST) + bb_r[...])
    A = jnp.dot(a * g, wc_r[...], preferred_element_type=jnp.float32,
                precision=lax.Precision.HIGHEST) + bc_r[...]
    a_out_r[...] = A
    rows = lax.broadcasted_iota(jnp.int32, (_BN, 1), 0) + i * _BN
    valid = rows < _N
    seg = lax.broadcasted_iota(jnp.int32, (_BN, _B), 1)
    mask = (bt_r[...] == seg) & valid
    vals = jnp.where(mask, A, -jnp.inf)
    blkmax = jnp.max(vals, axis=0, keepdims=True)

    @pl.when(i == 0)
    def _():
        m_out_r[...] = jnp.full((1, _B), -jnp.inf, jnp.float32)

    m_out_r[...] = jnp.maximum(m_out_r[...], blkmax)


def _att_call(h0, h1, h2, batch2, Wa, ba, Wb, bb, Wc, bc):
    return pl.pallas_call(
        _att_body,
        grid=(_G,),
        in_specs=[
            pl.BlockSpec((_BN, _H), lambda i: (i, 0)),
            pl.BlockSpec((_BN, _H), lambda i: (i, 0)),
            pl.BlockSpec((_BN, _H), lambda i: (i, 0)),
            pl.BlockSpec((_BN, 1), lambda i: (i, 0)),
            pl.BlockSpec((_H3, _H3), lambda i: (0, 0)),
            pl.BlockSpec((1, _H3), lambda i: (0, 0)),
            pl.BlockSpec((_H3, _H3), lambda i: (0, 0)),
            pl.BlockSpec((1, _H3), lambda i: (0, 0)),
            pl.BlockSpec((_H3, 1), lambda i: (0, 0)),
            pl.BlockSpec((1, 1), lambda i: (0, 0)),
        ],
        out_specs=[
            pl.BlockSpec((_BN, 1), lambda i: (i, 0)),
            pl.BlockSpec((1, _B), lambda i: (0, 0)),
        ],
        out_shape=[
            jax.ShapeDtypeStruct((_N, 1), jnp.float32),
            jax.ShapeDtypeStruct((1, _B), jnp.float32),
        ],
    )(h0, h1, h2, batch2, Wa, ba, Wb, bb, Wc, bc)


def _pool_body(h0_r, h1_r, h2_r, bt_r, a_r, m_r, wr_r, br_r, wk_r, bk_r,
               o_r, u_acc, s_acc):
    i = pl.program_id(0)

    @pl.when(i == 0)
    def _():
        u_acc[...] = jnp.zeros((_B, _H3), jnp.float32)
        s_acc[...] = jnp.zeros((_B, 1), jnp.float32)

    rows = lax.broadcasted_iota(jnp.int32, (_BN, 1), 0) + i * _BN
    valid = rows < _N
    hp = jnp.concatenate([h0_r[...], h1_r[...], h2_r[...]], axis=1)
    hp = jnp.where(valid, hp, 0.0)
    seg = lax.broadcasted_iota(jnp.int32, (_BN, _B), 1)
    mask = (bt_r[...] == seg) & valid
    e = jnp.where(mask, jnp.exp(a_r[...] - m_r[...]), 0.0)
    u_acc[...] += lax.dot_general(
        e, hp, (((0,), (0,)), ((), ())), preferred_element_type=jnp.float32, precision=lax.Precision.HIGHEST)
    s_acc[...] += lax.dot_general(
        e, jnp.ones((_BN, 1), jnp.float32), (((0,), (0,)), ((), ())),
        preferred_element_type=jnp.float32, precision=lax.Precision.HIGHEST)

    @pl.when(i == _G - 1)
    def _():
        s = s_acc[...]
        s_safe = jnp.where(s > 0.0, s, 1.0)
        pooled = u_acc[...] / s_safe
        hr = jnp.maximum(
            jnp.dot(pooled, wr_r[...], preferred_element_type=jnp.float32, precision=lax.Precision.HIGHEST)
            + br_r[...], 0.0)
        o_r[...] = (jnp.dot(hr, wk_r[...], preferred_element_type=jnp.float32, precision=lax.Precision.HIGHEST)
                    + bk_r[...])


def _pool_call(h0, h1, h2, batch2, A, m, Wr, br, Wk, bk):
    return pl.pallas_call(
        _pool_body,
        grid=(_G,),
        in_specs=[
            pl.BlockSpec((_BN, _H), lambda i: (i, 0)),
            pl.BlockSpec((_BN, _H), lambda i: (i, 0)),
            pl.BlockSpec((_BN, _H), lambda i: (i, 0)),
            pl.BlockSpec((_BN, 1), lambda i: (i, 0)),
            pl.BlockSpec((_BN, 1), lambda i: (i, 0)),
            pl.BlockSpec((1, _B), lambda i: (0, 0)),
            pl.BlockSpec((_H3, _H3), lambda i: (0, 0)),
            pl.BlockSpec((1, _H3), lambda i: (0, 0)),
            pl.BlockSpec((_H3, 1), lambda i: (0, 0)),
            pl.BlockSpec((1, 1), lambda i: (0, 0)),
        ],
        out_specs=pl.BlockSpec((_B, 1), lambda i: (0, 0)),
        out_shape=jax.ShapeDtypeStruct((_B, 1), jnp.float32),
        scratch_shapes=[
            pltpu.VMEM((_B, _H3), jnp.float32),
            pltpu.VMEM((_B, 1), jnp.float32),
        ],
    )(h0, h1, h2, batch2, A, m, Wr, br, Wk, bk)


def kernel(x, W_fc, b_fc, W1a, b1a, W1b, b1b, W2a, b2a, W2b, b2b,
           Wa, ba, Wb, bb, Wc, bc, Wr, br, Wk, bk, edge_index, batch):
    src = edge_index[0]
    dst = edge_index[1]
    pad = _EPAD - _E
    pi = jnp.arange(pad, dtype=jnp.int32)
    src_p = jnp.concatenate([src, pi % _N]).reshape(_EPAD // _K, _K)
    dst_p = jnp.concatenate([dst, _N + pi % (_NACC - _N)]).reshape(_EPAD // _K, _K)
    batch2 = batch.reshape(_N, 1)

    agg = _get_agg()
    h0 = _fc_call(x, W_fc, b_fc.reshape(1, _H))
    a00, a01 = agg(h0, src_p, dst_p)
    h1 = _gin_call(h0, a00, a01, W1a, b1a.reshape(1, _H), W1b, b1b.reshape(1, _H))
    a10, a11 = agg(h1, src_p, dst_p)
    h2 = _gin_call(h1, a10, a11, W2a, b2a.reshape(1, _H), W2b, b2b.reshape(1, _H))

    A, m = _att_call(h0, h1, h2, batch2,
                     Wa, ba.reshape(1, _H3), Wb, bb.reshape(1, _H3),
                     Wc, bc.reshape(1, 1))
    out = _pool_call(h0, h1, h2, batch2, A, m,
                     Wr, br.reshape(1, _H3), Wk, bk.reshape(1, 1))
    return out.reshape(-1)


# speedup vs baseline: 8.1921x; 1.1199x over previous
import functools

import jax
import jax.numpy as jnp
from jax import lax
from jax.experimental import pallas as pl
from jax.experimental.pallas import tpu as pltpu
from jax.experimental.pallas import tpu_sc as plsc

_N = 10000
_E = 320000
_B = 8
_DIN = 1792
_H = 128
_H3 = 384

_NACC = 10240
_NSUB = 16
_NCORE = 2
_EPW = 10240
_EPAD = _EPW * _NSUB * _NCORE
_K = 80
_NCH = _EPW // _K
_NQUAD = _NCH // 4
_RPS = _NACC // _NSUB
_RCH = 80
_NRC = _RPS // _RCH

_BN = 2048
_G = 5


def _build_agg():
    mesh = plsc.VectorSubcoreMesh(core_axis_name="c", subcore_axis_name="s")

    @functools.partial(
        pl.kernel,
        mesh=mesh,
        out_type=[
            jax.ShapeDtypeStruct((_NACC, _H), jnp.float32),
            jax.ShapeDtypeStruct((_NACC, _H), jnp.float32),
        ],
        scratch_types=[
            pltpu.VMEM((4, 1, _K), jnp.int32),
            pltpu.VMEM((4, 1, _K), jnp.int32),
            pltpu.VMEM((_K, _H), jnp.float32),
            pltpu.VMEM((_K, _H), jnp.float32),
            pltpu.VMEM((_K, _H), jnp.float32),
            pltpu.VMEM((_K, _H), jnp.float32),
            pltpu.VMEM_SHARED((_NACC, _H), jnp.float32),
            pltpu.SemaphoreType.DMA,
            pltpu.SemaphoreType.DMA,
            pltpu.SemaphoreType.DMA,
            pltpu.SemaphoreType.DMA,
            pltpu.SemaphoreType.DMA,
            pltpu.SemaphoreType.DMA,
            pltpu.SemaphoreType.DMA,
            pltpu.SemaphoreType.DMA,
        ],
    )
    def agg(h_hbm, src_hbm, dst_hbm, out0_hbm, out1_hbm,
            si, di, rowsa, rowsb, rowsc, rowsd, acc_sh,
            semi0, semi1, semi2, semi3, semg0, semg1, semg2, semg3):
        cid = lax.axis_index("c")
        sid = lax.axis_index("s")
        wid = cid * _NSUB + sid
        base = wid * _NCH
        rows = (rowsa, rowsb, rowsc, rowsd)
        semi = (semi0, semi1, semi2, semi3)
        semg = (semg0, semg1, semg2, semg3)

        def _ichunk(c, q):
            pltpu.async_copy(src_hbm.at[pl.ds(base + c, 1)], si.at[q], semi[q])
            pltpu.async_copy(dst_hbm.at[pl.ds(base + c, 1)], di.at[q], semi[q])

        def _iwait(c, q):
            pltpu.make_async_copy(src_hbm.at[pl.ds(base + c, 1)], si.at[q],
                                  semi[q]).wait()
            pltpu.make_async_copy(dst_hbm.at[pl.ds(base + c, 1)], di.at[q],
                                  semi[q]).wait()

        def _gstart(c, q):
            pltpu.async_copy(h_hbm.at[si.at[q, 0]], rows[q], semg[q])

        def _gwait(c, q):
            pltpu.make_async_copy(h_hbm.at[si.at[q, 0]], rows[q], semg[q]).wait()

        for u in range(4):
            _ichunk(u, u)

        def _zrow(r, carry):
            for cc in range(_H // 16):
                rowsd[r, pl.ds(cc * 16, 16)] = jnp.zeros((16,), jnp.float32)
            return carry

        lax.fori_loop(0, _RCH, _zrow, 0)
        for j in range(_NRC):
            pltpu.sync_copy(rowsd, acc_sh.at[pl.ds(sid * _RPS + j * _RCH, _RCH)])
        plsc.subcore_barrier()

        for u in range(3):
            _iwait(u, u)
            _gstart(u, u)

        def _quad(p, carry):
            for u in range(4):
                j = 4 * p + u
                q = u
                q3 = (u + 3) % 4
                _gwait(j, q)
                pltpu.sync_copy(rows[q], acc_sh.at[di.at[q, 0]], add=True)

                @pl.when(j + 4 < _NCH)
                def _():
                    _ichunk(j + 4, q)

                @pl.when(j + 3 < _NCH)
                def _():
                    _iwait(j + 3, q3)
                    _gstart(j + 3, q3)

            return carry

        lax.fori_loop(0, _NQUAD, _quad, 0)
        plsc.subcore_barrier()

        for j in range(_NRC):
            r0 = sid * _RPS + j * _RCH
            pltpu.sync_copy(acc_sh.at[pl.ds(r0, _RCH)], rowsd)

            @pl.when(cid == 0)
            def _():
                pltpu.sync_copy(rowsd, out0_hbm.at[pl.ds(r0, _RCH)])

            @pl.when(cid == 1)
            def _():
                pltpu.sync_copy(rowsd, out1_hbm.at[pl.ds(r0, _RCH)])

    return agg


_agg_cache = []


def _get_agg():
    if not _agg_cache:
        _agg_cache.append(_build_agg())
    return _agg_cache[0]


def _fc_body(x_r, w_r, b_r, o_r):
    o_r[...] = jnp.maximum(
        jnp.dot(x_r[...], w_r[...], preferred_element_type=jnp.float32) + b_r[...],
        0.0)


def _fc_call(x, W, b):
    return pl.pallas_call(
        _fc_body,
        grid=(_G,),
        in_specs=[
            pl.BlockSpec((_BN, _DIN), lambda i: (i, 0)),
            pl.BlockSpec((_DIN, _H), lambda i: (0, 0)),
            pl.BlockSpec((1, _H), lambda i: (0, 0)),
        ],
        out_specs=pl.BlockSpec((_BN, _H), lambda i: (i, 0)),
        out_shape=jax.ShapeDtypeStruct((_N, _H), jnp.float32),
    )(x, W, b)


def _gin_body(h_r, a0_r, a1_r, w1_r, b1_r, w2_r, b2_r, o_r):
    z = h_r[...] + a0_r[...] + a1_r[...]
    t = jnp.maximum(
        jnp.dot(z, w1_r[...], preferred_element_type=jnp.float32,
                precision=lax.Precision.HIGHEST) + b1_r[...], 0.0)
    o_r[...] = jnp.maximum(
        jnp.dot(t, w2_r[...], preferred_element_type=jnp.float32,
                precision=lax.Precision.HIGHEST) + b2_r[...], 0.0)


def _gin_call(h, a0, a1, W1, b1, W2, b2):
    return pl.pallas_call(
        _gin_body,
        grid=(_G,),
        in_specs=[
            pl.BlockSpec((_BN, _H), lambda i: (i, 0)),
            pl.BlockSpec((_BN, _H), lambda i: (i, 0)),
            pl.BlockSpec((_BN, _H), lambda i: (i, 0)),
            pl.BlockSpec((_H, _H), lambda i: (0, 0)),
            pl.BlockSpec((1, _H), lambda i: (0, 0)),
            pl.BlockSpec((_H, _H), lambda i: (0, 0)),
            pl.BlockSpec((1, _H), lambda i: (0, 0)),
        ],
        out_specs=pl.BlockSpec((_BN, _H), lambda i: (i, 0)),
        out_shape=jax.ShapeDtypeStruct((_N, _H), jnp.float32),
    )(h, a0, a1, W1, b1, W2, b2)


def _att_body(h0_r, h1_r, h2_r, bt_r, wa_r, ba_r, wb_r, bb_r, wc_r, bc_r,
              a_out_r, m_out_r):
    i = pl.program_id(0)
    hp = jnp.concatenate([h0_r[...], h1_r[...], h2_r[...]], axis=1)
    a = jnp.tanh(jnp.dot(hp, wa_r[...], preferred_element_type=jnp.float32,
                         precision=lax.Precision.HIGHEST) + ba_r[...])
    g = jax.nn.sigmoid(jnp.dot(hp, wb_r[...], preferred_element_type=jnp.float32,
                               precision=lax.Precision.HIGHEST) + bb_r[...])
    A = jnp.dot(a * g, wc_r[...], preferred_element_type=jnp.float32,
                precision=lax.Precision.HIGHEST) + bc_r[...]
    a_out_r[...] = A
    rows = lax.broadcasted_iota(jnp.int32, (_BN, 1), 0) + i * _BN
    valid = rows < _N
    seg = lax.broadcasted_iota(jnp.int32, (_BN, _B), 1)
    mask = (bt_r[...] == seg) & valid
    vals = jnp.where(mask, A, -jnp.inf)
    blkmax = jnp.max(vals, axis=0, keepdims=True)

    @pl.when(i == 0)
    def _():
        m_out_r[...] = jnp.full((1, _B), -jnp.inf, jnp.float32)

    m_out_r[...] = jnp.maximum(m_out_r[...], blkmax)


def _att_call(h0, h1, h2, batch2, Wa, ba, Wb, bb, Wc, bc):
    return pl.pallas_call(
        _att_body,
        grid=(_G,),
        in_specs=[
            pl.BlockSpec((_BN, _H), lambda i: (i, 0)),
            pl.BlockSpec((_BN, _H), lambda i: (i, 0)),
            pl.BlockSpec((_BN, _H), lambda i: (i, 0)),
            pl.BlockSpec((_BN, 1), lambda i: (i, 0)),
            pl.BlockSpec((_H3, _H3), lambda i: (0, 0)),
            pl.BlockSpec((1, _H3), lambda i: (0, 0)),
            pl.BlockSpec((_H3, _H3), lambda i: (0, 0)),
            pl.BlockSpec((1, _H3), lambda i: (0, 0)),
            pl.BlockSpec((_H3, 1), lambda i: (0, 0)),
            pl.BlockSpec((1, 1), lambda i: (0, 0)),
        ],
        out_specs=[
            pl.BlockSpec((_BN, 1), lambda i: (i, 0)),
            pl.BlockSpec((1, _B), lambda i: (0, 0)),
        ],
        out_shape=[
            jax.ShapeDtypeStruct((_N, 1), jnp.float32),
            jax.ShapeDtypeStruct((1, _B), jnp.float32),
        ],
    )(h0, h1, h2, batch2, Wa, ba, Wb, bb, Wc, bc)


def _pool_body(h0_r, h1_r, h2_r, bt_r, a_r, m_r, wr_r, br_r, wk_r, bk_r,
               o_r, u_acc, s_acc):
    i = pl.program_id(0)

    @pl.when(i == 0)
    def _():
        u_acc[...] = jnp.zeros((_B, _H3), jnp.float32)
        s_acc[...] = jnp.zeros((_B, 1), jnp.float32)

    rows = lax.broadcasted_iota(jnp.int32, (_BN, 1), 0) + i * _BN
    valid = rows < _N
    hp = jnp.concatenate([h0_r[...], h1_r[...], h2_r[...]], axis=1)
    hp = jnp.where(valid, hp, 0.0)
    seg = lax.broadcasted_iota(jnp.int32, (_BN, _B), 1)
    mask = (bt_r[...] == seg) & valid
    e = jnp.where(mask, jnp.exp(a_r[...] - m_r[...]), 0.0)
    u_acc[...] += lax.dot_general(
        e, hp, (((0,), (0,)), ((), ())), preferred_element_type=jnp.float32, precision=lax.Precision.HIGHEST)
    s_acc[...] += lax.dot_general(
        e, jnp.ones((_BN, 1), jnp.float32), (((0,), (0,)), ((), ())),
        preferred_element_type=jnp.float32, precision=lax.Precision.HIGHEST)

    @pl.when(i == _G - 1)
    def _():
        s = s_acc[...]
        s_safe = jnp.where(s > 0.0, s, 1.0)
        pooled = u_acc[...] / s_safe
        hr = jnp.maximum(
            jnp.dot(pooled, wr_r[...], preferred_element_type=jnp.float32, precision=lax.Precision.HIGHEST)
            + br_r[...], 0.0)
        o_r[...] = (jnp.dot(hr, wk_r[...], preferred_element_type=jnp.float32, precision=lax.Precision.HIGHEST)
                    + bk_r[...])


def _pool_call(h0, h1, h2, batch2, A, m, Wr, br, Wk, bk):
    return pl.pallas_call(
        _pool_body,
        grid=(_G,),
        in_specs=[
            pl.BlockSpec((_BN, _H), lambda i: (i, 0)),
            pl.BlockSpec((_BN, _H), lambda i: (i, 0)),
            pl.BlockSpec((_BN, _H), lambda i: (i, 0)),
            pl.BlockSpec((_BN, 1), lambda i: (i, 0)),
            pl.BlockSpec((_BN, 1), lambda i: (i, 0)),
            pl.BlockSpec((1, _B), lambda i: (0, 0)),
            pl.BlockSpec((_H3, _H3), lambda i: (0, 0)),
            pl.BlockSpec((1, _H3), lambda i: (0, 0)),
            pl.BlockSpec((_H3, 1), lambda i: (0, 0)),
            pl.BlockSpec((1, 1), lambda i: (0, 0)),
        ],
        out_specs=pl.BlockSpec((_B, 1), lambda i: (0, 0)),
        out_shape=jax.ShapeDtypeStruct((_B, 1), jnp.float32),
        scratch_shapes=[
            pltpu.VMEM((_B, _H3), jnp.float32),
            pltpu.VMEM((_B, 1), jnp.float32),
        ],
    )(h0, h1, h2, batch2, A, m, Wr, br, Wk, bk)


def kernel(x, W_fc, b_fc, W1a, b1a, W1b, b1b, W2a, b2a, W2b, b2b,
           Wa, ba, Wb, bb, Wc, bc, Wr, br, Wk, bk, edge_index, batch):
    src = edge_index[0]
    dst = edge_index[1]
    pad = _EPAD - _E
    pi = jnp.arange(pad, dtype=jnp.int32)
    src_p = jnp.concatenate([src, pi % _N]).reshape(_EPAD // _K, _K)
    dst_p = jnp.concatenate([dst, _N + pi % (_NACC - _N)]).reshape(_EPAD // _K, _K)
    batch2 = batch.reshape(_N, 1)

    agg = _get_agg()
    h0 = _fc_call(x, W_fc, b_fc.reshape(1, _H))
    a00, a01 = agg(h0, src_p, dst_p)
    h1 = _gin_call(h0, a00, a01, W1a, b1a.reshape(1, _H), W1b, b1b.reshape(1, _H))
    a10, a11 = agg(h1, src_p, dst_p)
    h2 = _gin_call(h1, a10, a11, W2a, b2a.reshape(1, _H), W2b, b2b.reshape(1, _H))

    A, m = _att_call(h0, h1, h2, batch2,
                     Wa, ba.reshape(1, _H3), Wb, bb.reshape(1, _H3),
                     Wc, bc.reshape(1, 1))
    out = _pool_call(h0, h1, h2, batch2, A, m,
                     Wr, br.reshape(1, _H3), Wk, bk.reshape(1, 1))
    return out.reshape(-1)
